# baseline (speedup 1.0000x reference)
import functools

import jax
import jax.numpy as jnp
from jax import lax
from jax.experimental import pallas as pl
from jax.experimental.pallas import tpu as pltpu

_SPB = 2


def _stats_kernel(x_ref, sum_ref, sq_ref, *, spb):
    x = x_ref[...]
    rows, c = x.shape
    x3 = x.reshape(spb, rows // spb, c)
    sum_ref[...] = jnp.sum(x3, axis=1)[None]
    sq_ref[...] = jnp.sum(x3 * x3, axis=1)[None]


def _apply_kernel(cls_ref, sum_ref, sq_ref, w1_ref, b1_ref, wg_ref, bg_ref,
                  wb_ref, bb_ref, x_ref, o_ref, *, inv_cnt, spb):
    i = pl.program_id(0)
    s = jnp.sum(sum_ref[...], axis=(0, 1))
    q = jnp.sum(sq_ref[...], axis=(0, 1))
    mean = (s * inv_cnt)[None, :]
    var = (q * inv_cnt)[None, :] - mean * mean
    rstd = lax.rsqrt(var + 1e-5)
    rows = [w1_ref[pl.ds(cls_ref[spb * i + j], 1), :] for j in range(spb)]
    h = jnp.maximum(jnp.concatenate(rows, axis=0) + b1_ref[...], 0.0)
    gamma = jnp.dot(h, wg_ref[...],
                    preferred_element_type=jnp.float32) + bg_ref[...]
    beta = jnp.dot(h, wb_ref[...],
                   preferred_element_type=jnp.float32) + bb_ref[...]
    scale = rstd * (1.0 + gamma)
    bias = beta - mean * scale
    x = x_ref[...]
    nrows, c = x.shape
    x3 = x.reshape(spb, nrows // spb, c)
    o_ref[...] = (x3 * scale[:, None, :]
                  + bias[:, None, :]).reshape(nrows, c)


@jax.jit
def _cond_batch_norm(x, cls_label, w1, b1, wg, bg, wb, bb):
    n, c, hgt, wid = x.shape
    hw = hgt * wid
    xr = x.astype(jnp.float32).transpose(0, 2, 3, 1).reshape(n * hw, c)

    spb = _SPB if n % _SPB == 0 else 1
    g = n // spb
    br = hw * spb
    sspb = 2 * spb if n % (2 * spb) == 0 else spb
    sg = n // sspb
    sbr = hw * sspb

    stats_spec = pl.BlockSpec((1, sspb, c), lambda i: (i, 0, 0))
    sum3, sq3 = pl.pallas_call(
        functools.partial(_stats_kernel, spb=sspb),
        out_shape=(jax.ShapeDtypeStruct((sg, sspb, c), jnp.float32),
                   jax.ShapeDtypeStruct((sg, sspb, c), jnp.float32)),
        grid=(sg,),
        in_specs=[pl.BlockSpec((sbr, c), lambda i: (i, 0))],
        out_specs=(stats_spec, stats_spec),
        compiler_params=pltpu.CompilerParams(
            dimension_semantics=("parallel",),
            vmem_limit_bytes=int(56 << 20)),
        cost_estimate=pl.CostEstimate(
            flops=3 * n * hw * c, transcendentals=0,
            bytes_accessed=n * hw * c * 4 + 8 * n * c),
    )(xr)

    constp = pl.BlockSpec((sg, sspb, c), lambda i, *_: (0, 0, 0))
    tilep = pl.BlockSpec((br, c), lambda i, *_: (i, 0))
    grid_spec = pltpu.PrefetchScalarGridSpec(
        num_scalar_prefetch=1,
        grid=(g,),
        in_specs=[constp, constp,
                  pl.BlockSpec(w1.shape, lambda i, *_: (0, 0)),
                  pl.BlockSpec(b1.shape, lambda i, *_: (0, 0)),
                  pl.BlockSpec(wg.shape, lambda i, *_: (0, 0)),
                  pl.BlockSpec(bg.shape, lambda i, *_: (0, 0)),
                  pl.BlockSpec(wb.shape, lambda i, *_: (0, 0)),
                  pl.BlockSpec(bb.shape, lambda i, *_: (0, 0)),
                  tilep],
        out_specs=tilep,
    )
    out2 = pl.pallas_call(
        functools.partial(_apply_kernel, inv_cnt=1.0 / float(n * hw),
                          spb=spb),
        out_shape=jax.ShapeDtypeStruct((n * hw, c), jnp.float32),
        grid_spec=grid_spec,
        compiler_params=pltpu.CompilerParams(
            dimension_semantics=("parallel",),
            vmem_limit_bytes=int(56 << 20)),
        cost_estimate=pl.CostEstimate(
            flops=2 * n * hw * c, transcendentals=c,
            bytes_accessed=2 * n * hw * c * 4 + 8 * n * c),
    )(cls_label, sum3, sq3, w1, b1, wg, bg, wb, bb, xr)
    return out2.reshape(n, hgt, wid, c).transpose(0, 3, 1, 2)


def kernel(x, cls_label, w1, b1, wg, bg, wb, bb):
    return _cond_batch_norm(x, cls_label, w1, b1, wg, bg, wb, bb)


# speedup vs baseline: 1.0228x; 1.0228x over previous
import functools

import jax
import jax.numpy as jnp
from jax import lax
from jax.experimental import pallas as pl
from jax.experimental.pallas import tpu as pltpu

_SPB = 2


def _stats_kernel(x_ref, sum_ref, sq_ref, *, spb):
    x = x_ref[...]
    rows, c = x.shape
    x3 = x.reshape(spb, rows // spb, c)
    sum_ref[...] = jnp.sum(x3, axis=1)[None]
    sq_ref[...] = jnp.sum(x3 * x3, axis=1)[None]


def _apply_kernel(cls_ref, sum_ref, sq_ref, w1_ref, b1_ref, wg_ref, bg_ref,
                  wb_ref, bb_ref, x_ref, o_ref, *, inv_cnt, spb):
    i = pl.program_id(0)
    s = jnp.sum(sum_ref[...], axis=(0, 1))
    q = jnp.sum(sq_ref[...], axis=(0, 1))
    mean = (s * inv_cnt)[None, :]
    var = (q * inv_cnt)[None, :] - mean * mean
    rstd = lax.rsqrt(var + 1e-5)
    rows = [w1_ref[pl.ds(cls_ref[spb * i + j], 1), :] for j in range(spb)]
    h = jnp.maximum(jnp.concatenate(rows, axis=0) + b1_ref[...], 0.0)
    gamma = jnp.dot(h, wg_ref[...],
                    preferred_element_type=jnp.float32) + bg_ref[...]
    beta = jnp.dot(h, wb_ref[...],
                   preferred_element_type=jnp.float32) + bb_ref[...]
    scale = rstd * (1.0 + gamma)
    bias = beta - mean * scale
    x = x_ref[...]
    nrows, c = x.shape
    x3 = x.reshape(spb, nrows // spb, c)
    o_ref[...] = (x3 * scale[:, None, :]
                  + bias[:, None, :]).reshape(nrows, c)


@jax.jit
def _cond_batch_norm(x, cls_label, w1, b1, wg, bg, wb, bb):
    n, c, hgt, wid = x.shape
    hw = hgt * wid
    xr = x.astype(jnp.float32).transpose(0, 2, 3, 1).reshape(n * hw, c)

    spb = _SPB if n % _SPB == 0 else 1
    g = n // spb
    br = hw * spb
    sspb = spb
    sg = n // sspb
    sbr = hw * sspb

    stats_spec = pl.BlockSpec((1, sspb, c), lambda i: (i, 0, 0))
    sum3, sq3 = pl.pallas_call(
        functools.partial(_stats_kernel, spb=sspb),
        out_shape=(jax.ShapeDtypeStruct((sg, sspb, c), jnp.float32),
                   jax.ShapeDtypeStruct((sg, sspb, c), jnp.float32)),
        grid=(sg,),
        in_specs=[pl.BlockSpec((sbr, c), lambda i: (i, 0))],
        out_specs=(stats_spec, stats_spec),
        compiler_params=pltpu.CompilerParams(
            dimension_semantics=("parallel",),
            vmem_limit_bytes=int(56 << 20)),
        cost_estimate=pl.CostEstimate(
            flops=3 * n * hw * c, transcendentals=0,
            bytes_accessed=n * hw * c * 4 + 8 * n * c),
    )(xr)

    constp = pl.BlockSpec((sg, sspb, c), lambda i, *_: (0, 0, 0))
    tilep = pl.BlockSpec((br, c), lambda i, *_: (i, 0))
    grid_spec = pltpu.PrefetchScalarGridSpec(
        num_scalar_prefetch=1,
        grid=(g,),
        in_specs=[constp, constp,
                  pl.BlockSpec(w1.shape, lambda i, *_: (0, 0)),
                  pl.BlockSpec(b1.shape, lambda i, *_: (0, 0)),
                  pl.BlockSpec(wg.shape, lambda i, *_: (0, 0)),
                  pl.BlockSpec(bg.shape, lambda i, *_: (0, 0)),
                  pl.BlockSpec(wb.shape, lambda i, *_: (0, 0)),
                  pl.BlockSpec(bb.shape, lambda i, *_: (0, 0)),
                  tilep],
        out_specs=tilep,
    )
    out2 = pl.pallas_call(
        functools.partial(_apply_kernel, inv_cnt=1.0 / float(n * hw),
                          spb=spb),
        out_shape=jax.ShapeDtypeStruct((n * hw, c), jnp.float32),
        grid_spec=grid_spec,
        compiler_params=pltpu.CompilerParams(
            dimension_semantics=("parallel",),
            vmem_limit_bytes=int(56 << 20)),
        cost_estimate=pl.CostEstimate(
            flops=2 * n * hw * c, transcendentals=c,
            bytes_accessed=2 * n * hw * c * 4 + 8 * n * c),
    )(cls_label, sum3, sq3, w1, b1, wg, bg, wb, bb, xr)
    return out2.reshape(n, hgt, wid, c).transpose(0, 3, 1, 2)


def kernel(x, cls_label, w1, b1, wg, bg, wb, bb):
    return _cond_batch_norm(x, cls_label, w1, b1, wg, bg, wb, bb)
